# baseline (speedup 1.0000x reference)
import jax
import jax.numpy as jnp
from jax import lax
from jax.experimental import pallas as pl
from jax.experimental.pallas import tpu as pltpu
from jax.experimental.pallas import tpu_sc as plsc

N = 25000
D = 128
E = 300000
NS = 16
NC = 2
CW = 128
CHUNKS = 152
SEGC = 8
NSEG = CHUNKS // SEGC
EPAD = NS * CHUNKS * CW
RPC = 12544
ACCR = 12560
DUMMY = RPC
ZR = ACCR // NS
OR = RPC // NS
NPAD = NC * RPC
BIG = 1 << 20

_mesh = plsc.VectorSubcoreMesh(core_axis_name="c", subcore_axis_name="s")


def _localize(lidx, lo, nchunks):
    def body(k, carry):
        for j in range(8):
            v = lidx[k, pl.ds(j * 16, 16)]
            inr = (v >= lo) & (v < lo + RPC)
            lidx[k, pl.ds(j * 16, 16)] = jnp.where(inr, v - lo, DUMMY)
        return carry
    lax.fori_loop(0, nchunks, body, 0)


def _fill2d(ref, cols, value):
    def body(k, carry):
        for j in range(cols // 16):
            ref[k, pl.ds(j * 16, 16)] = jnp.full((16,), value, ref.dtype)
        return carry
    lax.fori_loop(0, 128, body, 0)


def _zero_rows(acc, zbuf, base):
    off = 0
    for sz in (128,) * (ZR // 128) + (ZR % 128,):
        pltpu.sync_copy(zbuf.at[pl.ds(0, sz)], acc.at[pl.ds(base + off, sz)])
        off += sz


def _deg_body(i0, i1, i2, i3, out, dacc, rows, lidx):
    cid = lax.axis_index("c")
    sid = lax.axis_index("s")
    lo = cid * RPC
    for h, idx_hbm in enumerate((i0, i1, i2, i3)):
        _fill2d(rows, 128, 0.0)
        _zero_rows(dacc, rows, sid * ZR)
        _fill2d(rows, 128, 1.0)
        plsc.subcore_barrier()

        def seg_body(g, carry, idx_hbm=idx_hbm):
            pltpu.sync_copy(idx_hbm.at[sid, pl.ds(g * SEGC, SEGC)], lidx)
            _localize(lidx, lo, SEGC)

            def body(k, c2):
                pltpu.sync_copy(rows, dacc.at[lidx.at[k]], add=True)
                return c2
            lax.fori_loop(0, SEGC, body, 0)
            return carry
        lax.fori_loop(0, NSEG, seg_body, 0)
        plsc.subcore_barrier()
        pltpu.sync_copy(
            dacc.at[pl.ds(sid * OR, OR)],
            out.at[pl.ds(h * NPAD + cid * RPC + sid * OR, OR)])
        plsc.subcore_barrier()


_deg_call = pl.kernel(
    _deg_body,
    out_type=jax.ShapeDtypeStruct((4 * NPAD, D), jnp.float32),
    mesh=_mesh,
    scratch_types=[pltpu.VMEM_SHARED((ACCR, D), jnp.float32),
                   pltpu.VMEM((128, D), jnp.float32),
                   pltpu.VMEM((SEGC, CW), jnp.int32)],
)


GW = 192
GOPS = 102
TPE2 = GW * GOPS
EPAD2 = NS * TPE2


def _agg_body(p_dt, p_td, s_dt, t_dt, s_td, t_td, out_t, out_d,
              acc, sidx, lidx, rows, gsem):
    cid = lax.axis_index("c")
    sid = lax.axis_index("s")
    lo = cid * RPC
    for p_hbm, s_hbm, t_hbm, out in ((p_dt, s_dt, t_dt, out_t),
                                     (p_td, s_td, t_td, out_d)):
        _fill2d(rows, 128, 0.0)
        off = 0
        for sz in (128,) * (ZR // 128) + (ZR % 128,):
            pltpu.sync_copy(rows.at[pl.ds(0, sz)],
                            acc.at[pl.ds(sid * ZR + off, sz)])
            off += sz
        plsc.subcore_barrier()

        def body(k, carry, p_hbm=p_hbm, s_hbm=s_hbm, t_hbm=t_hbm):
            base = sid * TPE2 + k * GW
            pltpu.sync_copy(s_hbm.at[pl.ds(base, GW)], sidx)
            pltpu.sync_copy(t_hbm.at[pl.ds(base, GW)], lidx)

            def lzb(j, c2):
                v = lidx[pl.ds(j * 16, 16)]
                inr = (v >= lo) & (v < lo + RPC)
                lidx[pl.ds(j * 16, 16)] = jnp.where(inr, v - lo, DUMMY)
                return c2
            lax.fori_loop(0, GW // 16, lzb, 0)
            pltpu.async_copy(p_hbm.at[sidx], rows, gsem).wait()
            pltpu.sync_copy(rows, acc.at[lidx], add=True)
            return carry
        lax.fori_loop(0, GOPS, body, 0)
        plsc.subcore_barrier()
        pltpu.sync_copy(acc.at[pl.ds(sid * OR, OR)],
                        out.at[pl.ds(cid * RPC + sid * OR, OR)])
        plsc.subcore_barrier()


_agg_call = pl.kernel(
    _agg_body,
    out_type=[jax.ShapeDtypeStruct((NPAD, D), jnp.float32)] * 2,
    mesh=_mesh,
    scratch_types=[pltpu.VMEM_SHARED((ACCR, D), jnp.float32),
                   pltpu.VMEM((GW,), jnp.int32),
                   pltpu.VMEM((GW,), jnp.int32),
                   pltpu.VMEM((GW, D), jnp.float32),
                   pltpu.SemaphoreType.DMA],
)


BLK = 200
GRID = N // BLK


NB = NC * RPC


def _norm(c):
    return lax.rsqrt(jnp.maximum(c, 1.0))


def _mm_body(f_ref, c_ref, w_ref, o_ref):
    o_ref[...] = jnp.dot(f_ref[...] * _norm(c_ref[...]), w_ref[...],
                         preferred_element_type=jnp.float32)


def _scale_mm(feat, cnt, w):
    return pl.pallas_call(
        _mm_body,
        grid=(GRID,),
        in_specs=[pl.BlockSpec((BLK, D), lambda i: (i, 0)),
                  pl.BlockSpec((BLK, 1), lambda i: (i, 0)),
                  pl.BlockSpec((D, D), lambda i: (0, 0))],
        out_specs=pl.BlockSpec((BLK, D), lambda i: (i, 0)),
        out_shape=jax.ShapeDtypeStruct((N, D), jnp.float32),
    )(feat, cnt, w)


def _mid_body(a_ref, cd_ref, b_ref, cs_ref, w_ref, o_ref):
    h = jax.nn.relu(a_ref[...] * _norm(cd_ref[...]) + b_ref[...])
    o_ref[...] = jnp.dot(h * _norm(cs_ref[...]), w_ref[...],
                         preferred_element_type=jnp.float32)


def _mid_mm(agg, cnt_dst, b, cnt_src2, w2):
    return pl.pallas_call(
        _mid_body,
        grid=(GRID,),
        in_specs=[pl.BlockSpec((BLK, D), lambda i: (i, 0)),
                  pl.BlockSpec((BLK, 1), lambda i: (i, 0)),
                  pl.BlockSpec((1, D), lambda i: (0, 0)),
                  pl.BlockSpec((BLK, 1), lambda i: (i, 0)),
                  pl.BlockSpec((D, D), lambda i: (0, 0))],
        out_specs=pl.BlockSpec((BLK, D), lambda i: (i, 0)),
        out_shape=jax.ShapeDtypeStruct((N, D), jnp.float32),
    )(agg, cnt_dst, b, cnt_src2, w2)


def _fin_body(a_ref, cd_ref, b_ref, o_ref):
    o_ref[...] = a_ref[...] * _norm(cd_ref[...]) + b_ref[...]


def _finish(agg, cnt_dst, b):
    return pl.pallas_call(
        _fin_body,
        grid=(GRID,),
        in_specs=[pl.BlockSpec((BLK, D), lambda i: (i, 0)),
                  pl.BlockSpec((BLK, 1), lambda i: (i, 0)),
                  pl.BlockSpec((1, D), lambda i: (0, 0))],
        out_specs=pl.BlockSpec((BLK, D), lambda i: (i, 0)),
        out_shape=jax.ShapeDtypeStruct((N, D), jnp.float32),
    )(agg, cnt_dst, b)


def _pad_edges(idx, fill):
    pad = jnp.full((EPAD - E,), fill, jnp.int32)
    return jnp.concatenate([idx, pad]).reshape(NS, CHUNKS, CW)


def _pad_flat2(idx, fill):
    pad = jnp.full((EPAD2 - E,), fill, jnp.int32)
    return jnp.concatenate([idx, pad])


def kernel(drug_feat, target_feat, edge_dt, edge_td,
           W1_dt, b1_dt, W1_td, b1_td, W2_dt, b2_dt, W2_td, b2_td):
    src_dt, dst_dt = edge_dt[0], edge_dt[1]
    src_td, dst_td = edge_td[0], edge_td[1]
    sd_dt = _pad_edges(src_dt, BIG)
    sd_td = _pad_edges(src_td, BIG)
    td_dt = _pad_edges(dst_dt, BIG)
    td_td = _pad_edges(dst_td, BIG)
    sga_dt = _pad_flat2(src_dt, 0)
    sga_td = _pad_flat2(src_td, 0)
    tda_dt = _pad_flat2(dst_dt, BIG)
    tda_td = _pad_flat2(dst_td, BIG)
    b1_dt2, b1_td2 = b1_dt.reshape(1, D), b1_td.reshape(1, D)
    b2_dt2, b2_td2 = b2_dt.reshape(1, D), b2_td.reshape(1, D)

    deg = _deg_call(sd_dt, td_dt, sd_td, td_td)

    def _cnt(h):
        return deg[h * NPAD:(h + 1) * NPAD, :1]
    h0, h1, h2, h3 = _cnt(0), _cnt(1), _cnt(2), _cnt(3)

    p1_dt = _scale_mm(drug_feat, h0, W1_dt)
    p1_td = _scale_mm(target_feat, h2, W1_td)
    agg_t1, agg_d1 = _agg_call(p1_dt, p1_td, sga_dt, tda_dt, sga_td, tda_td)

    p2_dt = _mid_mm(agg_d1, h3, b1_td2, h0, W2_dt)
    p2_td = _mid_mm(agg_t1, h1, b1_dt2, h2, W2_td)
    agg_t2, agg_d2 = _agg_call(p2_dt, p2_td, sga_dt, tda_dt, sga_td, tda_td)

    out_target = _finish(agg_t2, h1, b2_dt2)
    out_drug = _finish(agg_d2, h3, b2_td2)
    return (out_drug, out_target)


# speedup vs baseline: 1.1152x; 1.1152x over previous
import jax
import jax.numpy as jnp
from jax import lax
from jax.experimental import pallas as pl
from jax.experimental.pallas import tpu as pltpu
from jax.experimental.pallas import tpu_sc as plsc

N = 25000
D = 128
E = 300000
NS = 16
NC = 2
CW = 128
CHUNKS = 152
SEGC = 8
NSEG = CHUNKS // SEGC
EPAD = NS * CHUNKS * CW
RPC = 12544
ACCR = 12560
DUMMY = RPC
ZR = ACCR // NS
OR = RPC // NS
NPAD = NC * RPC
BIG = 1 << 20

_mesh = plsc.VectorSubcoreMesh(core_axis_name="c", subcore_axis_name="s")


def _localize(lidx, lo, nchunks):
    def body(k, carry):
        for j in range(8):
            v = lidx[k, pl.ds(j * 16, 16)]
            inr = (v >= lo) & (v < lo + RPC)
            lidx[k, pl.ds(j * 16, 16)] = jnp.where(inr, v - lo, DUMMY)
        return carry
    lax.fori_loop(0, nchunks, body, 0)


def _fill2d(ref, cols, value):
    def body(k, carry):
        for j in range(cols // 16):
            ref[k, pl.ds(j * 16, 16)] = jnp.full((16,), value, ref.dtype)
        return carry
    lax.fori_loop(0, 128, body, 0)


def _zero_rows(acc, zbuf, base):
    off = 0
    for sz in (128,) * (ZR // 128) + (ZR % 128,):
        pltpu.sync_copy(zbuf.at[pl.ds(0, sz)], acc.at[pl.ds(base + off, sz)])
        off += sz


def _deg_body(i0, i1, i2, i3, out, dacc, rows, lidx):
    cid = lax.axis_index("c")
    sid = lax.axis_index("s")
    lo = cid * RPC
    for h, idx_hbm in enumerate((i0, i1, i2, i3)):
        _fill2d(rows, 128, 0.0)
        _zero_rows(dacc, rows, sid * ZR)
        _fill2d(rows, 128, 1.0)
        plsc.subcore_barrier()

        def seg_body(g, carry, idx_hbm=idx_hbm):
            pltpu.sync_copy(idx_hbm.at[sid, pl.ds(g * SEGC, SEGC)], lidx)
            _localize(lidx, lo, SEGC)

            def body(k, c2):
                pltpu.sync_copy(rows, dacc.at[lidx.at[k]], add=True)
                return c2
            lax.fori_loop(0, SEGC, body, 0)
            return carry
        lax.fori_loop(0, NSEG, seg_body, 0)
        plsc.subcore_barrier()
        pltpu.sync_copy(
            dacc.at[pl.ds(sid * OR, OR)],
            out.at[pl.ds(h * NPAD + cid * RPC + sid * OR, OR)])
        plsc.subcore_barrier()


_deg_call = pl.kernel(
    _deg_body,
    out_type=jax.ShapeDtypeStruct((4 * NPAD, D), jnp.float32),
    mesh=_mesh,
    scratch_types=[pltpu.VMEM_SHARED((ACCR, D), jnp.float32),
                   pltpu.VMEM((128, D), jnp.float32),
                   pltpu.VMEM((SEGC, CW), jnp.int32)],
)


def _agg_body(p_dt, p_td, s_dt, t_dt, s_td, t_td, out_t, out_d,
              acc, sidx, lidx, rows, gsem):
    cid = lax.axis_index("c")
    sid = lax.axis_index("s")
    lo = cid * RPC
    for p_hbm, s_hbm, t_hbm, out in ((p_dt, s_dt, t_dt, out_t),
                                     (p_td, s_td, t_td, out_d)):
        _fill2d(rows, 128, 0.0)
        _zero_rows(acc, rows, sid * ZR)
        plsc.subcore_barrier()

        def seg_body(g, carry, p_hbm=p_hbm, s_hbm=s_hbm, t_hbm=t_hbm):
            pltpu.sync_copy(s_hbm.at[sid, pl.ds(g * SEGC, SEGC)], sidx)
            pltpu.sync_copy(t_hbm.at[sid, pl.ds(g * SEGC, SEGC)], lidx)
            _localize(lidx, lo, SEGC)

            def body(k, c2):
                pltpu.async_copy(p_hbm.at[sidx.at[k]], rows, gsem).wait()
                pltpu.sync_copy(rows, acc.at[lidx.at[k]], add=True)
                return c2
            lax.fori_loop(0, SEGC, body, 0)
            return carry
        lax.fori_loop(0, NSEG, seg_body, 0)
        plsc.subcore_barrier()
        pltpu.sync_copy(acc.at[pl.ds(sid * OR, OR)],
                        out.at[pl.ds(cid * RPC + sid * OR, OR)])
        plsc.subcore_barrier()


_agg_call = pl.kernel(
    _agg_body,
    out_type=[jax.ShapeDtypeStruct((NPAD, D), jnp.float32)] * 2,
    mesh=_mesh,
    scratch_types=[pltpu.VMEM_SHARED((ACCR, D), jnp.float32),
                   pltpu.VMEM((SEGC, CW), jnp.int32),
                   pltpu.VMEM((SEGC, CW), jnp.int32),
                   pltpu.VMEM((CW, D), jnp.float32),
                   pltpu.SemaphoreType.DMA],
)


BLK = 200
GRID = N // BLK


NB = NC * RPC


def _norm(c):
    return lax.rsqrt(jnp.maximum(c, 1.0))


def _mm_body(f_ref, c_ref, w_ref, o_ref):
    o_ref[...] = jnp.dot(f_ref[...] * _norm(c_ref[...]), w_ref[...],
                         preferred_element_type=jnp.float32)


def _scale_mm(feat, cnt, w):
    return pl.pallas_call(
        _mm_body,
        grid=(GRID,),
        in_specs=[pl.BlockSpec((BLK, D), lambda i: (i, 0)),
                  pl.BlockSpec((BLK, 1), lambda i: (i, 0)),
                  pl.BlockSpec((D, D), lambda i: (0, 0))],
        out_specs=pl.BlockSpec((BLK, D), lambda i: (i, 0)),
        out_shape=jax.ShapeDtypeStruct((N, D), jnp.float32),
    )(feat, cnt, w)


def _mid_body(a_ref, cd_ref, b_ref, cs_ref, w_ref, o_ref):
    h = jax.nn.relu(a_ref[...] * _norm(cd_ref[...]) + b_ref[...])
    o_ref[...] = jnp.dot(h * _norm(cs_ref[...]), w_ref[...],
                         preferred_element_type=jnp.float32)


def _mid_mm(agg, cnt_dst, b, cnt_src2, w2):
    return pl.pallas_call(
        _mid_body,
        grid=(GRID,),
        in_specs=[pl.BlockSpec((BLK, D), lambda i: (i, 0)),
                  pl.BlockSpec((BLK, 1), lambda i: (i, 0)),
                  pl.BlockSpec((1, D), lambda i: (0, 0)),
                  pl.BlockSpec((BLK, 1), lambda i: (i, 0)),
                  pl.BlockSpec((D, D), lambda i: (0, 0))],
        out_specs=pl.BlockSpec((BLK, D), lambda i: (i, 0)),
        out_shape=jax.ShapeDtypeStruct((N, D), jnp.float32),
    )(agg, cnt_dst, b, cnt_src2, w2)


def _fin_body(a_ref, cd_ref, b_ref, o_ref):
    o_ref[...] = a_ref[...] * _norm(cd_ref[...]) + b_ref[...]


def _finish(agg, cnt_dst, b):
    return pl.pallas_call(
        _fin_body,
        grid=(GRID,),
        in_specs=[pl.BlockSpec((BLK, D), lambda i: (i, 0)),
                  pl.BlockSpec((BLK, 1), lambda i: (i, 0)),
                  pl.BlockSpec((1, D), lambda i: (0, 0))],
        out_specs=pl.BlockSpec((BLK, D), lambda i: (i, 0)),
        out_shape=jax.ShapeDtypeStruct((N, D), jnp.float32),
    )(agg, cnt_dst, b)


def _pad_edges(idx, fill):
    pad = jnp.full((EPAD - E,), fill, jnp.int32)
    return jnp.concatenate([idx, pad]).reshape(NS, CHUNKS, CW)


def kernel(drug_feat, target_feat, edge_dt, edge_td,
           W1_dt, b1_dt, W1_td, b1_td, W2_dt, b2_dt, W2_td, b2_td):
    src_dt, dst_dt = edge_dt[0], edge_dt[1]
    src_td, dst_td = edge_td[0], edge_td[1]
    sg_dt = _pad_edges(src_dt, 0)
    sg_td = _pad_edges(src_td, 0)
    sd_dt = _pad_edges(src_dt, BIG)
    sd_td = _pad_edges(src_td, BIG)
    td_dt = _pad_edges(dst_dt, BIG)
    td_td = _pad_edges(dst_td, BIG)
    b1_dt2, b1_td2 = b1_dt.reshape(1, D), b1_td.reshape(1, D)
    b2_dt2, b2_td2 = b2_dt.reshape(1, D), b2_td.reshape(1, D)

    deg = _deg_call(sd_dt, td_dt, sd_td, td_td)

    def _cnt(h):
        return deg[h * NPAD:(h + 1) * NPAD, :1]
    h0, h1, h2, h3 = _cnt(0), _cnt(1), _cnt(2), _cnt(3)

    p1_dt = _scale_mm(drug_feat, h0, W1_dt)
    p1_td = _scale_mm(target_feat, h2, W1_td)
    agg_t1, agg_d1 = _agg_call(p1_dt, p1_td, sg_dt, td_dt, sg_td, td_td)

    p2_dt = _mid_mm(agg_d1, h3, b1_td2, h0, W2_dt)
    p2_td = _mid_mm(agg_t1, h1, b1_dt2, h2, W2_td)
    agg_t2, agg_d2 = _agg_call(p2_dt, p2_td, sg_dt, td_dt, sg_td, td_td)

    out_target = _finish(agg_t2, h1, b2_dt2)
    out_drug = _finish(agg_d2, h3, b2_td2)
    return (out_drug, out_target)
